# baseline (speedup 1.0000x reference)
import numpy as np
import jax
import jax.numpy as jnp
from jax import lax
from jax.experimental import pallas as pl
from jax.experimental.pallas import tpu as pltpu
from jax.experimental.pallas import tpu_sc as plsc

_RES1 = 511.0
_P1 = 2654435761
_P2 = 805459861
_TMASK = 2**22 - 1

_NC, _NS = 2, 16
_NW = _NC * _NS
_C = 2048
_S = 512
_CP = 2048
_HB = 512


def _xyz_sel() -> np.ndarray:
    m = np.zeros((384, 384), np.float32)
    for l in range(128):
        m[3 * l, l] = 1.0
        m[3 * l + 1, 128 + l] = 1.0
        m[3 * l + 2, 256 + l] = 1.0
    return m


_XYZ_M = _xyz_sel()


def _hash_tc(x_ref, y_ref, z_ref, o_ref):
    def p2i(v):
        v = jnp.minimum(jnp.maximum(v, -1.0), 1.0)
        v = (v + 1.0) / 2.0
        v = v * _RES1
        return v.astype(jnp.int32).astype(jnp.uint32)

    h = (p2i(x_ref[...]) ^ (p2i(y_ref[...]) * jnp.uint32(_P1))
         ^ (p2i(z_ref[...]) * jnp.uint32(_P2)))
    h = h & jnp.uint32(_TMASK)
    o_ref[...] = h.astype(jnp.int32)


def _sc_expand(tabf_hbm, out_hbm, pairs, bld, sem):
    t2 = tabf_hbm.shape[0]
    t_w = t2 // 2 // _NW
    n_ch = t_w // _CP
    wid = lax.axis_index("s") * _NC + lax.axis_index("c")
    r0_w = wid * t_w
    lane01 = lax.iota(jnp.int32, 16) & 1

    def chunk(i, carry):
        r0 = r0_w + i * _CP
        pltpu.sync_copy(tabf_hbm.at[pl.ds(2 * r0, 2 * _CP)], pairs)

        def group(g, carry):
            p = pairs[pl.ds(g * 16, 16)]
            for k in range(8):
                row = jnp.take_along_axis(p, lane01 + 2 * k, axis=0)
                bld[g * 8 + k, :] = row
            return carry

        lax.fori_loop(0, _CP // 8, group, 0)
        pltpu.sync_copy(bld, out_hbm.at[pl.ds(r0, _CP)])
        return carry

    lax.fori_loop(0, n_ch, chunk, 0)


def _sc_gather(idx_hbm, table_hbm, out_hbm,
               idx_a, idx_b, rows_a, rows_b, obuf_a, obuf_b, sem_a, sem_b):
    n = idx_hbm.shape[0]
    n_w = n // _NW
    n_chunks = n_w // _C
    wid = lax.axis_index("s") * _NC + lax.axis_index("c")
    w_base = wid * n_w
    bufs = [(idx_a, rows_a, obuf_a, sem_a), (idx_b, rows_b, obuf_b, sem_b)]
    lane = lax.iota(jnp.int32, 16)
    masks = [(lane >= 2 * k) & (lane < 2 * k + 2) for k in range(8)]
    lo_half = lane < 8
    perm_e = (lane % 8) * 2
    perm_o = perm_e + 1

    def fire(idxbuf, rows, sem):
        return [
            pltpu.async_copy(table_hbm.at[idxbuf.at[pl.ds(j * _S, _S)]],
                             rows.at[pl.ds(j * _S, _S)], sem)
            for j in range(_C // _S)
        ]

    def extract(rows, obuf):
        def group(g2, carry):
            accs = []
            for half in range(2):
                vs = [rows[g2 * 16 + half * 8 + k, :] for k in range(8)]
                acc = vs[7]
                for k in range(6, -1, -1):
                    acc = jnp.where(masks[k], vs[k], acc)
                accs.append(acc)
            a_e = jnp.take_along_axis(accs[0], perm_e, axis=0)
            b_e = jnp.take_along_axis(accs[1], perm_e, axis=0)
            a_o = jnp.take_along_axis(accs[0], perm_o, axis=0)
            b_o = jnp.take_along_axis(accs[1], perm_o, axis=0)
            f0 = jnp.where(lo_half, a_e, b_e)
            f1 = jnp.where(lo_half, a_o, b_o)
            blk = g2 // 8
            off = (g2 % 8) * 16
            obuf[pl.ds(blk * 256 + off, 16)] = f0
            obuf[pl.ds(blk * 256 + 128 + off, 16)] = f1
            return carry

        lax.fori_loop(0, _C // 16, group, 0)

    idx0, rows0, _, sem0 = bufs[0]
    pltpu.sync_copy(idx_hbm.at[pl.ds(w_base, _C)], idx0)
    handles = fire(idx0, rows0, sem0)
    for i in range(n_chunks):
        cur_idx, cur_rows, cur_obuf, cur_sem = bufs[i % 2]
        nxt_idx, nxt_rows, nxt_obuf, nxt_sem = bufs[(i + 1) % 2]
        if i + 1 < n_chunks:
            pltpu.sync_copy(idx_hbm.at[pl.ds(w_base + (i + 1) * _C, _C)],
                            nxt_idx)
            nxt_handles = fire(nxt_idx, nxt_rows, nxt_sem)
        else:
            nxt_handles = None
        for hnd in handles:
            hnd.wait()
        extract(cur_rows, cur_obuf)
        pltpu.sync_copy(cur_obuf,
                        out_hbm.at[pl.ds(2 * (w_base + i * _C), 2 * _C)])
        handles = nxt_handles


def kernel(X, table):
    n = X.shape[0]
    t = table.shape[0]
    f = table.shape[1]

    hb = _HB * 128
    nb = n // hb
    xt = X.T.reshape(3 * n)
    idx = pl.pallas_call(
        _hash_tc,
        grid=(nb,),
        in_specs=[
            pl.BlockSpec((hb,), lambda i: (i,)),
            pl.BlockSpec((hb,), lambda i: (i + nb,)),
            pl.BlockSpec((hb,), lambda i: (i + 2 * nb,)),
        ],
        out_specs=pl.BlockSpec((hb,), lambda i: (i,)),
        out_shape=jax.ShapeDtypeStruct((n,), jnp.int32),
    )(xt, xt, xt)

    mesh = plsc.VectorSubcoreMesh(core_axis_name="c", subcore_axis_name="s")
    sc_params = pltpu.CompilerParams(use_tc_tiling_on_sc=False)

    table16 = pl.kernel(
        _sc_expand,
        out_type=jax.ShapeDtypeStruct((t, 16), jnp.float32),
        mesh=mesh,
        scratch_types=[
            pltpu.VMEM((2 * _CP,), jnp.float32),
            pltpu.VMEM((_CP, 16), jnp.float32),
            pltpu.SemaphoreType.DMA,
        ],
        compiler_params=sc_params,
    )(table.reshape(2 * t))

    out = pl.kernel(
        _sc_gather,
        out_type=jax.ShapeDtypeStruct((2 * n,), jnp.float32),
        mesh=mesh,
        scratch_types=[
            pltpu.VMEM((_C,), jnp.int32),
            pltpu.VMEM((_C,), jnp.int32),
            pltpu.VMEM((_C, 16), jnp.float32),
            pltpu.VMEM((_C, 16), jnp.float32),
            pltpu.VMEM((2 * _C,), jnp.float32),
            pltpu.VMEM((2 * _C,), jnp.float32),
            pltpu.SemaphoreType.DMA,
            pltpu.SemaphoreType.DMA,
        ],
        compiler_params=sc_params,
    )(idx, table16)
    return out.reshape(n // 128, 2, 128).transpose(0, 2, 1).reshape(n, f)


# speedup vs baseline: 9.8377x; 9.8377x over previous
import numpy as np
import jax
import jax.numpy as jnp
from jax import lax
from jax.experimental import pallas as pl
from jax.experimental.pallas import tpu as pltpu
from jax.experimental.pallas import tpu_sc as plsc

_RES1 = 511.0
_P1 = 2654435761
_P2 = 805459861
_TMASK = 2**22 - 1

_NC, _NS = 2, 16
_NW = _NC * _NS
_C = 2048
_S = 512
_CP = 2048
_HB = 512


def _xyz_sel() -> np.ndarray:
    m = np.zeros((384, 384), np.float32)
    for l in range(128):
        m[3 * l, l] = 1.0
        m[3 * l + 1, 128 + l] = 1.0
        m[3 * l + 2, 256 + l] = 1.0
    return m


_XYZ_M = _xyz_sel()


def _hash_tc(x_ref, y_ref, z_ref, o_ref):
    def p2i(v):
        v = jnp.minimum(jnp.maximum(v, -1.0), 1.0)
        v = (v + 1.0) / 2.0
        v = v * _RES1
        return v.astype(jnp.int32).astype(jnp.uint32)

    h = (p2i(x_ref[...]) ^ (p2i(y_ref[...]) * jnp.uint32(_P1))
         ^ (p2i(z_ref[...]) * jnp.uint32(_P2)))
    h = h & jnp.uint32(_TMASK)
    o_ref[...] = h.astype(jnp.int32)


def _sc_expand(tabp_hbm, out_hbm, pairs, bld, sem):
    t2 = tabp_hbm.shape[0]
    t = t2 // 2
    t_w = t // _NW
    n_ch = t_w // _CP
    wid = lax.axis_index("s") * _NC + lax.axis_index("c")
    r0_w = wid * t_w
    lane = lax.iota(jnp.int32, 16)
    lane01 = lane & 1
    even = lane01 == 0
    half = lax.shift_right_logical(lane, 1)

    for i in range(n_ch):
        r0 = r0_w + i * _CP
        pltpu.sync_copy(tabp_hbm.at[pl.ds(r0, _CP)], pairs.at[pl.ds(0, _CP)])
        pltpu.sync_copy(tabp_hbm.at[pl.ds(t + r0, _CP)],
                        pairs.at[pl.ds(_CP, _CP)])

        def group(g, carry):
            f0v = pairs[pl.ds(g * 16, 16)]
            f1v = pairs[pl.ds(_CP + g * 16, 16)]
            p0 = jnp.where(even,
                           jnp.take_along_axis(f0v, half, axis=0),
                           jnp.take_along_axis(f1v, half, axis=0))
            p1 = jnp.where(even,
                           jnp.take_along_axis(f0v, half + 8, axis=0),
                           jnp.take_along_axis(f1v, half + 8, axis=0))
            for k in range(8):
                bld[g * 16 + k, :] = jnp.take_along_axis(
                    p0, lane01 + 2 * k, axis=0)
                bld[g * 16 + 8 + k, :] = jnp.take_along_axis(
                    p1, lane01 + 2 * k, axis=0)
            return carry

        lax.fori_loop(0, _CP // 16, group, 0)
        pltpu.sync_copy(bld, out_hbm.at[pl.ds(r0, _CP)])


def _sc_gather(idx_hbm, table_hbm, out_hbm,
               idx_a, idx_b, rows_a, rows_b, obuf_a, obuf_b, sem_a, sem_b):
    n = idx_hbm.shape[0]
    n_w = n // _NW
    n_chunks = n_w // _C
    wid = lax.axis_index("s") * _NC + lax.axis_index("c")
    w_base = wid * n_w
    bufs = [(idx_a, rows_a, obuf_a, sem_a), (idx_b, rows_b, obuf_b, sem_b)]
    lane = lax.iota(jnp.int32, 16)
    masks = [(lane >= 2 * k) & (lane < 2 * k + 2) for k in range(8)]
    lo_half = lane < 8
    perm_e = (lane % 8) * 2
    perm_o = perm_e + 1

    def fire(idxbuf, rows, sem):
        return [
            pltpu.async_copy(table_hbm.at[idxbuf.at[pl.ds(j * _S, _S)]],
                             rows.at[pl.ds(j * _S, _S)], sem)
            for j in range(_C // _S)
        ]

    def extract(rows, obuf):
        def group(g2, carry):
            accs = []
            for half in range(2):
                vs = [rows[g2 * 16 + half * 8 + k, :] for k in range(8)]
                acc = vs[7]
                for k in range(6, -1, -1):
                    acc = jnp.where(masks[k], vs[k], acc)
                accs.append(acc)
            a_e = jnp.take_along_axis(accs[0], perm_e, axis=0)
            b_e = jnp.take_along_axis(accs[1], perm_e, axis=0)
            a_o = jnp.take_along_axis(accs[0], perm_o, axis=0)
            b_o = jnp.take_along_axis(accs[1], perm_o, axis=0)
            f0 = jnp.where(lo_half, a_e, b_e)
            f1 = jnp.where(lo_half, a_o, b_o)
            blk = g2 // 8
            off = (g2 % 8) * 16
            obuf[pl.ds(blk * 256 + off, 16)] = f0
            obuf[pl.ds(blk * 256 + 128 + off, 16)] = f1
            return carry

        lax.fori_loop(0, _C // 16, group, 0)

    idx0, rows0, _, sem0 = bufs[0]
    pltpu.sync_copy(idx_hbm.at[pl.ds(w_base, _C)], idx0)
    handles = fire(idx0, rows0, sem0)
    for i in range(n_chunks):
        cur_idx, cur_rows, cur_obuf, cur_sem = bufs[i % 2]
        nxt_idx, nxt_rows, nxt_obuf, nxt_sem = bufs[(i + 1) % 2]
        if i + 1 < n_chunks:
            pltpu.sync_copy(idx_hbm.at[pl.ds(w_base + (i + 1) * _C, _C)],
                            nxt_idx)
            nxt_handles = fire(nxt_idx, nxt_rows, nxt_sem)
        else:
            nxt_handles = None
        for hnd in handles:
            hnd.wait()
        extract(cur_rows, cur_obuf)
        pltpu.sync_copy(cur_obuf,
                        out_hbm.at[pl.ds(2 * (w_base + i * _C), 2 * _C)])
        handles = nxt_handles


def kernel(X, table):
    n = X.shape[0]
    t = table.shape[0]
    f = table.shape[1]

    hb = _HB * 128
    nb = n // hb
    xt = X.T.reshape(3 * n)
    idx = pl.pallas_call(
        _hash_tc,
        grid=(nb,),
        in_specs=[
            pl.BlockSpec((hb,), lambda i: (i,)),
            pl.BlockSpec((hb,), lambda i: (i + nb,)),
            pl.BlockSpec((hb,), lambda i: (i + 2 * nb,)),
        ],
        out_specs=pl.BlockSpec((hb,), lambda i: (i,)),
        out_shape=jax.ShapeDtypeStruct((n,), jnp.int32),
    )(xt, xt, xt)

    mesh = plsc.VectorSubcoreMesh(core_axis_name="c", subcore_axis_name="s")
    sc_params = pltpu.CompilerParams(use_tc_tiling_on_sc=False)

    table16 = pl.kernel(
        _sc_expand,
        out_type=jax.ShapeDtypeStruct((t, 16), jnp.float32),
        mesh=mesh,
        scratch_types=[
            pltpu.VMEM((2 * _CP,), jnp.float32),
            pltpu.VMEM((_CP, 16), jnp.float32),
            pltpu.SemaphoreType.DMA,
        ],
        compiler_params=sc_params,
    )(table.T.reshape(2 * t))

    out = pl.kernel(
        _sc_gather,
        out_type=jax.ShapeDtypeStruct((2 * n,), jnp.float32),
        mesh=mesh,
        scratch_types=[
            pltpu.VMEM((_C,), jnp.int32),
            pltpu.VMEM((_C,), jnp.int32),
            pltpu.VMEM((_C, 16), jnp.float32),
            pltpu.VMEM((_C, 16), jnp.float32),
            pltpu.VMEM((2 * _C,), jnp.float32),
            pltpu.VMEM((2 * _C,), jnp.float32),
            pltpu.SemaphoreType.DMA,
            pltpu.SemaphoreType.DMA,
        ],
        compiler_params=sc_params,
    )(idx, table16)
    return out.reshape(n // 128, 2, 128).transpose(0, 2, 1).reshape(n, f)


# speedup vs baseline: 9.8537x; 1.0016x over previous
import numpy as np
import jax
import jax.numpy as jnp
from jax import lax
from jax.experimental import pallas as pl
from jax.experimental.pallas import tpu as pltpu
from jax.experimental.pallas import tpu_sc as plsc

_RES1 = 511.0
_P1 = 2654435761
_P2 = 805459861
_TMASK = 2**22 - 1

_NC, _NS = 2, 16
_NW = _NC * _NS
_C = 2048
_S = 512
_CP = 2048
_HB = 512


def _xyz_sel() -> np.ndarray:
    m = np.zeros((384, 384), np.float32)
    for l in range(128):
        m[3 * l, l] = 1.0
        m[3 * l + 1, 128 + l] = 1.0
        m[3 * l + 2, 256 + l] = 1.0
    return m


_XYZ_M = _xyz_sel()


def _hash_tc(x_ref, y_ref, z_ref, o_ref):
    def p2i(v):
        v = jnp.minimum(jnp.maximum(v, -1.0), 1.0)
        v = (v + 1.0) / 2.0
        v = v * _RES1
        return v.astype(jnp.int32).astype(jnp.uint32)

    h = (p2i(x_ref[...]) ^ (p2i(y_ref[...]) * jnp.uint32(_P1))
         ^ (p2i(z_ref[...]) * jnp.uint32(_P2)))
    h = h & jnp.uint32(_TMASK)
    o_ref[...] = h.astype(jnp.int32)


def _sc_expand(tabp_hbm, out_hbm, pairs, bld_a, bld_b, sem):
    bld = [bld_a, bld_b]
    t2 = tabp_hbm.shape[0]
    t = t2 // 2
    t_w = t // _NW
    n_ch = t_w // _CP
    wid = lax.axis_index("s") * _NC + lax.axis_index("c")
    r0_w = wid * t_w
    lane = lax.iota(jnp.int32, 16)
    lane01 = lane & 1
    even = lane01 == 0
    half = lax.shift_right_logical(lane, 1)

    prev = None
    for i in range(n_ch):
        r0 = r0_w + i * _CP
        pltpu.sync_copy(tabp_hbm.at[pl.ds(r0, _CP)], pairs.at[pl.ds(0, _CP)])
        pltpu.sync_copy(tabp_hbm.at[pl.ds(t + r0, _CP)],
                        pairs.at[pl.ds(_CP, _CP)])
        cur_bld = bld[i % 2]

        def group(g, carry, cur_bld=cur_bld):
            f0v = pairs[pl.ds(g * 16, 16)]
            f1v = pairs[pl.ds(_CP + g * 16, 16)]
            p0 = jnp.where(even,
                           jnp.take_along_axis(f0v, half, axis=0),
                           jnp.take_along_axis(f1v, half, axis=0))
            p1 = jnp.where(even,
                           jnp.take_along_axis(f0v, half + 8, axis=0),
                           jnp.take_along_axis(f1v, half + 8, axis=0))
            for k in range(8):
                cur_bld[g * 16 + k, :] = jnp.take_along_axis(
                    p0, lane01 + 2 * k, axis=0)
                cur_bld[g * 16 + 8 + k, :] = jnp.take_along_axis(
                    p1, lane01 + 2 * k, axis=0)
            return carry

        lax.fori_loop(0, _CP // 16, group, 0)
        if prev is not None:
            prev.wait()
        prev = pltpu.async_copy(cur_bld, out_hbm.at[pl.ds(r0, _CP)], sem)
    prev.wait()


def _sc_gather(idx_hbm, table_hbm, out_hbm,
               idx_a, idx_b, rows_a, rows_b, obuf_a, obuf_b, sem_a, sem_b):
    n = idx_hbm.shape[0]
    n_w = n // _NW
    n_chunks = n_w // _C
    wid = lax.axis_index("s") * _NC + lax.axis_index("c")
    w_base = wid * n_w
    bufs = [(idx_a, rows_a, obuf_a, sem_a), (idx_b, rows_b, obuf_b, sem_b)]
    lane = lax.iota(jnp.int32, 16)
    masks = [(lane >= 2 * k) & (lane < 2 * k + 2) for k in range(8)]
    lo_half = lane < 8
    perm_e = (lane % 8) * 2
    perm_o = perm_e + 1

    def fire(idxbuf, rows, sem):
        return [
            pltpu.async_copy(table_hbm.at[idxbuf.at[pl.ds(j * _S, _S)]],
                             rows.at[pl.ds(j * _S, _S)], sem)
            for j in range(_C // _S)
        ]

    def extract(rows, obuf):
        def group(g2, carry):
            accs = []
            for half in range(2):
                vs = [rows[g2 * 16 + half * 8 + k, :] for k in range(8)]
                acc = vs[7]
                for k in range(6, -1, -1):
                    acc = jnp.where(masks[k], vs[k], acc)
                accs.append(acc)
            a_e = jnp.take_along_axis(accs[0], perm_e, axis=0)
            b_e = jnp.take_along_axis(accs[1], perm_e, axis=0)
            a_o = jnp.take_along_axis(accs[0], perm_o, axis=0)
            b_o = jnp.take_along_axis(accs[1], perm_o, axis=0)
            f0 = jnp.where(lo_half, a_e, b_e)
            f1 = jnp.where(lo_half, a_o, b_o)
            blk = g2 // 8
            off = (g2 % 8) * 16
            obuf[pl.ds(blk * 256 + off, 16)] = f0
            obuf[pl.ds(blk * 256 + 128 + off, 16)] = f1
            return carry

        lax.fori_loop(0, _C // 16, group, 0)

    idx0, rows0, _, sem0 = bufs[0]
    pltpu.sync_copy(idx_hbm.at[pl.ds(w_base, _C)], idx0)
    handles = fire(idx0, rows0, sem0)
    for i in range(n_chunks):
        cur_idx, cur_rows, cur_obuf, cur_sem = bufs[i % 2]
        nxt_idx, nxt_rows, nxt_obuf, nxt_sem = bufs[(i + 1) % 2]
        if i + 1 < n_chunks:
            pltpu.sync_copy(idx_hbm.at[pl.ds(w_base + (i + 1) * _C, _C)],
                            nxt_idx)
            nxt_handles = fire(nxt_idx, nxt_rows, nxt_sem)
        else:
            nxt_handles = None
        for hnd in handles:
            hnd.wait()
        extract(cur_rows, cur_obuf)
        pltpu.sync_copy(cur_obuf,
                        out_hbm.at[pl.ds(2 * (w_base + i * _C), 2 * _C)])
        handles = nxt_handles


def kernel(X, table):
    n = X.shape[0]
    t = table.shape[0]
    f = table.shape[1]

    hb = _HB * 128
    nb = n // hb
    xt = X.T.reshape(3 * n)
    idx = pl.pallas_call(
        _hash_tc,
        grid=(nb,),
        in_specs=[
            pl.BlockSpec((hb,), lambda i: (i,)),
            pl.BlockSpec((hb,), lambda i: (i + nb,)),
            pl.BlockSpec((hb,), lambda i: (i + 2 * nb,)),
        ],
        out_specs=pl.BlockSpec((hb,), lambda i: (i,)),
        out_shape=jax.ShapeDtypeStruct((n,), jnp.int32),
    )(xt, xt, xt)

    mesh = plsc.VectorSubcoreMesh(core_axis_name="c", subcore_axis_name="s")
    sc_params = pltpu.CompilerParams(use_tc_tiling_on_sc=False)

    table16 = pl.kernel(
        _sc_expand,
        out_type=jax.ShapeDtypeStruct((t, 16), jnp.float32),
        mesh=mesh,
        scratch_types=[
            pltpu.VMEM((2 * _CP,), jnp.float32),
            pltpu.VMEM((_CP, 16), jnp.float32),
            pltpu.VMEM((_CP, 16), jnp.float32),
            pltpu.SemaphoreType.DMA,
        ],
        compiler_params=sc_params,
    )(table.T.reshape(2 * t))

    out = pl.kernel(
        _sc_gather,
        out_type=jax.ShapeDtypeStruct((2 * n,), jnp.float32),
        mesh=mesh,
        scratch_types=[
            pltpu.VMEM((_C,), jnp.int32),
            pltpu.VMEM((_C,), jnp.int32),
            pltpu.VMEM((_C, 16), jnp.float32),
            pltpu.VMEM((_C, 16), jnp.float32),
            pltpu.VMEM((2 * _C,), jnp.float32),
            pltpu.VMEM((2 * _C,), jnp.float32),
            pltpu.SemaphoreType.DMA,
            pltpu.SemaphoreType.DMA,
        ],
        compiler_params=sc_params,
    )(idx, table16)
    return out.reshape(n // 128, 2, 128).transpose(0, 2, 1).reshape(n, f)
